# baseline (speedup 1.0000x reference)
import jax
import jax.numpy as jnp
import numpy as np
from jax import lax
from jax.experimental import pallas as pl
from jax.experimental.pallas import tpu as pltpu
from jax.experimental.pallas import tpu_sc as plsc

CLIP = 15.0
N = 1024
NST = 10
ITER = 5
K = 512
B = 1024
L = 16
NV = N // L
NWORKERS = 32
ROWS_PER_W = B // NWORKERS

_I32_SIGN = np.int32(-2**31)


def _f(a, b):
    s = (plsc.bitcast(a, jnp.int32) ^ plsc.bitcast(b, jnp.int32)) & _I32_SIGN
    m = jnp.minimum(jnp.abs(a), jnp.abs(b))
    return plsc.bitcast(plsc.bitcast(m, jnp.int32) | s, jnp.float32)


def _f2(a_abs, a_bits, b):
    s = (a_bits ^ plsc.bitcast(b, jnp.int32)) & _I32_SIGN
    m = jnp.minimum(a_abs, jnp.abs(b))
    return plsc.bitcast(plsc.bitcast(m, jnp.int32) | s, jnp.float32)


def _clip(v):
    return jnp.clip(v, -CLIP, CLIP)


def _decode_body(rx_hbm, info_hbm, frozen_hbm,
                 uall_hbm, ufin_hbm,
                 right_v, left_v, rinit_v, info_v, fg_v, fgx_v, uo_v, rxp_v,
                 rx_sem, u_sem):
    wid = lax.axis_index("s") * 2 + lax.axis_index("c")
    row0 = wid * ROWS_PER_W

    pltpu.sync_copy(frozen_hbm, right_v.at[pl.ds(0, N)])
    pltpu.sync_copy(info_hbm, info_v)

    @plsc.parallel_loop(0, K // L, unroll=4)
    def _(k):
        base = k * L
        idx = info_v[pl.ds(base, L)]
        fg_v[pl.ds(base, L)] = plsc.load_gather(right_v, [idx])
        fgx_v[pl.ds(base, L)] = plsc.load_gather(right_v, [idx ^ 1])

    def stage_lo(j, rref, roff, oref, ooff, a_is_left, first=False):
        h = 1 << j
        loff = (j + 1) * N
        W = 2 * L
        lane = lax.iota(jnp.int32, L)
        pat = ((lane >> j) << (j + 1)) | (lane & (h - 1))
        pat_b = pat + h

        @plsc.parallel_loop(0, NV // 2, unroll=2)
        def _(i):
            rwin = rref.at[pl.ds(roff + i * W, W)]
            owin = oref.at[pl.ds(ooff + i * W, W)]
            rt = plsc.load_gather(rwin, [pat])
            rb = plsc.load_gather(rwin, [pat_b])
            if first:
                plsc.store_scatter(owin, [pat], _f(rt, rb))
                plsc.store_scatter(owin, [pat_b], rb)
                return
            lwin = left_v.at[pl.ds(loff + i * W, W)]
            lt = plsc.load_gather(lwin, [pat])
            lb = plsc.load_gather(lwin, [pat_b])
            if a_is_left:
                a_abs, a_bits = jnp.abs(lt), plsc.bitcast(lt, jnp.int32)
                nt = _f2(a_abs, a_bits, lb + rb)
                nb = _clip(_f2(a_abs, a_bits, rt) + lb)
            else:
                a_abs, a_bits = jnp.abs(rt), plsc.bitcast(rt, jnp.int32)
                nt = _f2(a_abs, a_bits, lb + rb)
                nb = _clip(_f2(a_abs, a_bits, lt) + rb)
            plsc.store_scatter(owin, [pat], nt)
            plsc.store_scatter(owin, [pat_b], nb)

    def stage_hi(j, rref, roff, oref, ooff, a_is_left, first=False):
        h = 1 << j
        hs = h // L
        loff = (j + 1) * N
        unb = a_is_left and j == NST - 1

        @plsc.parallel_loop(0, NV // 2, unroll=4)
        def _(i):
            tv = (i & ~(hs - 1)) * 2 + (i & (hs - 1))
            pt = tv * L
            pb = pt + h
            rt = rref[pl.ds(roff + pt, L)]
            rb = rref[pl.ds(roff + pb, L)]
            if first:
                oref[pl.ds(ooff + pt, L)] = _f(rt, rb)
                oref[pl.ds(ooff + pb, L)] = rb
                return
            lt = left_v[pl.ds(loff + pt, L)]
            lb = left_v[pl.ds(loff + pb, L)]
            if a_is_left:
                a_abs, a_bits = jnp.abs(lt), plsc.bitcast(lt, jnp.int32)
                nt = _f2(a_abs, a_bits, lb + rb)
                if unb:
                    nt = _clip(nt)
                nb = _clip(_f2(a_abs, a_bits, rt) + lb)
            else:
                a_abs, a_bits = jnp.abs(rt), plsc.bitcast(rt, jnp.int32)
                nt = _f2(a_abs, a_bits, lb + rb)
                nb = _clip(_f2(a_abs, a_bits, lt) + rb)
            oref[pl.ds(ooff + pt, L)] = nt
            oref[pl.ds(ooff + pb, L)] = nb

    def stage(j, rref, roff, oref, ooff, a_is_left, first=False):
        fn = stage_lo if (1 << j) < L else stage_hi
        fn(j, rref, roff, oref, ooff, a_is_left, first=first)

    stage(0, right_v, 0, rinit_v, 0, False, first=True)
    for j in range(1, NST - 1):
        stage(j, rinit_v, (j - 1) * N, rinit_v, j * N, False, first=True)

    def right_sweep():
        for j in range(NST - 1):
            stage(j, right_v, j * N, right_v, (j + 1) * N, False)

    def left_sweep(iter1):
        for j in range(NST - 1, 0, -1):
            if iter1:
                stage(j, rinit_v, (j - 1) * N, left_v, j * N, True)
            else:
                stage(j, right_v, j * N, left_v, j * N, True)

    def emit_u(it, row):
        @plsc.parallel_loop(0, K // L, unroll=4)
        def _(k):
            base = k * L
            idx = info_v[pl.ds(base, L)]
            l1i = plsc.load_gather(left_v, [N + idx])
            l1x = plsc.load_gather(left_v, [N + (idx ^ 1)])
            fx = fgx_v[pl.ds(base, L)]
            m = (idx & 1) == 0
            t = _f(l1i, l1x + fx)
            b = _f(l1x, fx) + l1i
            left0 = jnp.where(m, t, _clip(b))
            uo_v[pl.ds(it * K + base, L)] = left0 + fg_v[pl.ds(base, L)]
        pltpu.async_copy(uo_v.at[pl.ds(it * K, K)], uall_hbm.at[it, row], u_sem)

    pltpu.async_copy(rx_hbm.at[row0], rxp_v, rx_sem)

    def row_body(rr, c0):
        row = row0 + rr
        pltpu.make_async_copy(rx_hbm.at[row], rxp_v, rx_sem).wait()

        @plsc.parallel_loop(0, NV, unroll=4)
        def _(k):
            base = k * L
            left_v[pl.ds(NST * N + base, L)] = rxp_v[pl.ds(base, L)]

        @pl.when(rr + 1 < ROWS_PER_W)
        def _():
            pltpu.async_copy(rx_hbm.at[row + 1], rxp_v, rx_sem)

        left_sweep(iter1=True)
        emit_u(0, row)

        def iter_body(it, c):
            right_sweep()
            left_sweep(iter1=False)
            emit_u(it, row)
            return c
        lax.fori_loop(1, ITER, iter_body, 0)

        pltpu.sync_copy(uo_v.at[pl.ds((ITER - 1) * K, K)], ufin_hbm.at[row])
        for itx in range(ITER):
            pltpu.make_async_copy(
                uo_v.at[pl.ds(itx * K, K)], uall_hbm.at[0, row], u_sem).wait()
        return c0

    lax.fori_loop(0, ROWS_PER_W, row_body, 0)


@jax.jit
def _decode(rx, info_idx, frozen):
    mesh = plsc.VectorSubcoreMesh(core_axis_name="c", subcore_axis_name="s")
    fn = pl.kernel(
        _decode_body,
        out_type=(
            jax.ShapeDtypeStruct((ITER, B, K), jnp.float32),
            jax.ShapeDtypeStruct((B, K), jnp.float32),
        ),
        mesh=mesh,
        compiler_params=pltpu.CompilerParams(needs_layout_passes=False),
        scratch_types=[
            pltpu.VMEM((NST * N,), jnp.float32),
            pltpu.VMEM(((NST + 1) * N,), jnp.float32),
            pltpu.VMEM(((NST - 1) * N,), jnp.float32),
            pltpu.VMEM((K,), jnp.int32),
            pltpu.VMEM((K,), jnp.float32),
            pltpu.VMEM((K,), jnp.float32),
            pltpu.VMEM((ITER * K,), jnp.float32),
            pltpu.VMEM((N,), jnp.float32),
            pltpu.SemaphoreType.DMA,
            pltpu.SemaphoreType.DMA,
        ],
    )
    return fn(rx, info_idx, frozen)


def kernel(rx, info_idx, info_mask):
    frozen = (1.0 - info_mask) * CLIP
    u_all, u_final = _decode(rx, info_idx, frozen)
    not_satisfied = jnp.arange(B, dtype=jnp.int32)
    return (u_all, u_final, not_satisfied)


# speedup vs baseline: 1.0007x; 1.0007x over previous
import jax
import jax.numpy as jnp
import numpy as np
from jax import lax
from jax.experimental import pallas as pl
from jax.experimental.pallas import tpu as pltpu
from jax.experimental.pallas import tpu_sc as plsc

CLIP = 15.0
N = 1024
NST = 10
ITER = 5
K = 512
B = 1024
L = 16
NV = N // L
NWORKERS = 32
ROWS_PER_W = B // NWORKERS

_I32_SIGN = np.int32(-2**31)


def _f(a, b):
    s = (plsc.bitcast(a, jnp.int32) ^ plsc.bitcast(b, jnp.int32)) & _I32_SIGN
    m = jnp.minimum(jnp.abs(a), jnp.abs(b))
    return plsc.bitcast(plsc.bitcast(m, jnp.int32) | s, jnp.float32)


def _f2(a_abs, a_bits, b):
    s = (a_bits ^ plsc.bitcast(b, jnp.int32)) & _I32_SIGN
    m = jnp.minimum(a_abs, jnp.abs(b))
    return plsc.bitcast(plsc.bitcast(m, jnp.int32) | s, jnp.float32)


def _clip(v):
    return jnp.clip(v, -CLIP, CLIP)


def _decode_body(rx_hbm, info_hbm, frozen_hbm,
                 uall_hbm, ufin_hbm,
                 right_v, left_v, rinit_v, info_v, fg_v, fgx_v, uo_v, rxp_v,
                 rx_sem, u_sem):
    wid = lax.axis_index("s") * 2 + lax.axis_index("c")
    row0 = wid * ROWS_PER_W

    pltpu.sync_copy(frozen_hbm, right_v.at[pl.ds(0, N)])
    pltpu.sync_copy(info_hbm, info_v)

    @plsc.parallel_loop(0, K // L, unroll=4)
    def _(k):
        base = k * L
        idx = info_v[pl.ds(base, L)]
        fg_v[pl.ds(base, L)] = plsc.load_gather(right_v, [idx])
        fgx_v[pl.ds(base, L)] = plsc.load_gather(right_v, [idx ^ 1])

    def stage_lo(j, rref, roff, oref, ooff, a_is_left, first=False):
        h = 1 << j
        loff = (j + 1) * N
        W = 2 * L
        lane = lax.iota(jnp.int32, L)
        pat = ((lane >> j) << (j + 1)) | (lane & (h - 1))
        pat_b = pat + h

        @plsc.parallel_loop(0, NV // 2, unroll=2)
        def _(i):
            rwin = rref.at[pl.ds(roff + i * W, W)]
            owin = oref.at[pl.ds(ooff + i * W, W)]
            rt = plsc.load_gather(rwin, [pat])
            rb = plsc.load_gather(rwin, [pat_b])
            if first:
                plsc.store_scatter(owin, [pat], _f(rt, rb))
                plsc.store_scatter(owin, [pat_b], rb)
                return
            lwin = left_v.at[pl.ds(loff + i * W, W)]
            lt = plsc.load_gather(lwin, [pat])
            lb = plsc.load_gather(lwin, [pat_b])
            if a_is_left:
                a_abs, a_bits = jnp.abs(lt), plsc.bitcast(lt, jnp.int32)
                nt = _f2(a_abs, a_bits, lb + rb)
                nb = _clip(_f2(a_abs, a_bits, rt) + lb)
            else:
                a_abs, a_bits = jnp.abs(rt), plsc.bitcast(rt, jnp.int32)
                nt = _f2(a_abs, a_bits, lb + rb)
                nb = _clip(_f2(a_abs, a_bits, lt) + rb)
            plsc.store_scatter(owin, [pat], nt)
            plsc.store_scatter(owin, [pat_b], nb)

    def stage_hi(j, rref, roff, oref, ooff, a_is_left, first=False):
        h = 1 << j
        hs = h // L
        loff = (j + 1) * N
        unb = a_is_left and j == NST - 1

        @plsc.parallel_loop(0, NV // 2, unroll=2)
        def _(i):
            tv = (i & ~(hs - 1)) * 2 + (i & (hs - 1))
            pt = tv * L
            pb = pt + h
            rt = rref[pl.ds(roff + pt, L)]
            rb = rref[pl.ds(roff + pb, L)]
            if first:
                oref[pl.ds(ooff + pt, L)] = _f(rt, rb)
                oref[pl.ds(ooff + pb, L)] = rb
                return
            lt = left_v[pl.ds(loff + pt, L)]
            lb = left_v[pl.ds(loff + pb, L)]
            if a_is_left:
                a_abs, a_bits = jnp.abs(lt), plsc.bitcast(lt, jnp.int32)
                nt = _f2(a_abs, a_bits, lb + rb)
                if unb:
                    nt = _clip(nt)
                nb = _clip(_f2(a_abs, a_bits, rt) + lb)
            else:
                a_abs, a_bits = jnp.abs(rt), plsc.bitcast(rt, jnp.int32)
                nt = _f2(a_abs, a_bits, lb + rb)
                nb = _clip(_f2(a_abs, a_bits, lt) + rb)
            oref[pl.ds(ooff + pt, L)] = nt
            oref[pl.ds(ooff + pb, L)] = nb

    def stage(j, rref, roff, oref, ooff, a_is_left, first=False):
        fn = stage_lo if (1 << j) < L else stage_hi
        fn(j, rref, roff, oref, ooff, a_is_left, first=first)

    stage(0, right_v, 0, rinit_v, 0, False, first=True)
    for j in range(1, NST - 1):
        stage(j, rinit_v, (j - 1) * N, rinit_v, j * N, False, first=True)

    def right_sweep():
        for j in range(NST - 1):
            stage(j, right_v, j * N, right_v, (j + 1) * N, False)

    def left_sweep(iter1):
        for j in range(NST - 1, 0, -1):
            if iter1:
                stage(j, rinit_v, (j - 1) * N, left_v, j * N, True)
            else:
                stage(j, right_v, j * N, left_v, j * N, True)

    def emit_u(it, row):
        @plsc.parallel_loop(0, K // L, unroll=4)
        def _(k):
            base = k * L
            idx = info_v[pl.ds(base, L)]
            l1i = plsc.load_gather(left_v, [N + idx])
            l1x = plsc.load_gather(left_v, [N + (idx ^ 1)])
            fx = fgx_v[pl.ds(base, L)]
            m = (idx & 1) == 0
            t = _f(l1i, l1x + fx)
            b = _f(l1x, fx) + l1i
            left0 = jnp.where(m, t, _clip(b))
            uo_v[pl.ds(it * K + base, L)] = left0 + fg_v[pl.ds(base, L)]
        pltpu.async_copy(uo_v.at[pl.ds(it * K, K)], uall_hbm.at[it, row], u_sem)

    pltpu.async_copy(rx_hbm.at[row0], rxp_v, rx_sem)

    def row_body(rr, c0):
        row = row0 + rr
        pltpu.make_async_copy(rx_hbm.at[row], rxp_v, rx_sem).wait()

        @plsc.parallel_loop(0, NV, unroll=4)
        def _(k):
            base = k * L
            left_v[pl.ds(NST * N + base, L)] = rxp_v[pl.ds(base, L)]

        @pl.when(rr + 1 < ROWS_PER_W)
        def _():
            pltpu.async_copy(rx_hbm.at[row + 1], rxp_v, rx_sem)

        left_sweep(iter1=True)
        emit_u(0, row)

        def iter_body(it, c):
            right_sweep()
            left_sweep(iter1=False)
            emit_u(it, row)
            return c
        lax.fori_loop(1, ITER, iter_body, 0)

        pltpu.sync_copy(uo_v.at[pl.ds((ITER - 1) * K, K)], ufin_hbm.at[row])
        for itx in range(ITER):
            pltpu.make_async_copy(
                uo_v.at[pl.ds(itx * K, K)], uall_hbm.at[0, row], u_sem).wait()
        return c0

    lax.fori_loop(0, ROWS_PER_W, row_body, 0)


@jax.jit
def _decode(rx, info_idx, frozen):
    mesh = plsc.VectorSubcoreMesh(core_axis_name="c", subcore_axis_name="s")
    fn = pl.kernel(
        _decode_body,
        out_type=(
            jax.ShapeDtypeStruct((ITER, B, K), jnp.float32),
            jax.ShapeDtypeStruct((B, K), jnp.float32),
        ),
        mesh=mesh,
        compiler_params=pltpu.CompilerParams(needs_layout_passes=False),
        scratch_types=[
            pltpu.VMEM((NST * N,), jnp.float32),
            pltpu.VMEM(((NST + 1) * N,), jnp.float32),
            pltpu.VMEM(((NST - 1) * N,), jnp.float32),
            pltpu.VMEM((K,), jnp.int32),
            pltpu.VMEM((K,), jnp.float32),
            pltpu.VMEM((K,), jnp.float32),
            pltpu.VMEM((ITER * K,), jnp.float32),
            pltpu.VMEM((N,), jnp.float32),
            pltpu.SemaphoreType.DMA,
            pltpu.SemaphoreType.DMA,
        ],
    )
    return fn(rx, info_idx, frozen)


def kernel(rx, info_idx, info_mask):
    frozen = (1.0 - info_mask) * CLIP
    u_all, u_final = _decode(rx, info_idx, frozen)
    not_satisfied = jnp.arange(B, dtype=jnp.int32)
    return (u_all, u_final, not_satisfied)
